# baseline (speedup 1.0000x reference)
import functools

import jax
import jax.numpy as jnp
from jax import lax
from jax.experimental import pallas as pl
from jax.experimental.pallas import tpu as pltpu
from jax.experimental.pallas import tpu_sc as plsc

_NUM_CORES = 2
_NUM_SUBCORES = 16
_NUM_WORKERS = _NUM_CORES * _NUM_SUBCORES
_CHUNK = 128
_LANES = 16


def _gather_body(n_chunks, dim, idx_hbm, table_hbm, out_hbm, idx_v, rows_v,
                 buf_v, sems):
    wid = lax.axis_index("s") * _NUM_CORES + lax.axis_index("c")
    pltpu.sync_copy(idx_hbm.at[wid], idx_v)
    copies = [
        pltpu.async_copy(
            table_hbm.at[idx_v.at[j]], rows_v.at[pl.ds(j * _CHUNK, _CHUNK)],
            sems[j],
        )
        for j in range(n_chunks)
    ]
    iota = lax.iota(jnp.int32, _LANES)
    n_lc = _CHUNK // _LANES
    for j in range(n_chunks):
        copies[j].wait()
        idx0 = [
            jnp.full((_LANES,), j * _CHUNK + lc * _LANES, jnp.int32) + iota
            for lc in range(n_lc)
        ]
        for c in range(dim):
            cvec = jnp.full((_LANES,), c, jnp.int32)
            for lc in range(n_lc):
                v = plsc.load_gather(rows_v, [idx0[lc], cvec])
                buf_v[c // 8, j, c % 8, pl.ds(lc * _LANES, _LANES)] = v
    j0 = wid * n_chunks
    for i in range(dim // 8):
        pltpu.sync_copy(buf_v.at[i], out_hbm.at[i, pl.ds(j0, n_chunks)])


def kernel(fuel_id, table):
    (batch,) = fuel_id.shape
    _, dim = table.shape
    b_per_w = batch // _NUM_WORKERS
    n_chunks = b_per_w // _CHUNK
    idx = fuel_id.astype(jnp.int32).reshape(_NUM_WORKERS, n_chunks, _CHUNK)

    gather = pl.kernel(
        functools.partial(_gather_body, n_chunks, dim),
        out_type=jax.ShapeDtypeStruct(
            (dim // 8, batch // _CHUNK, 8, _CHUNK), jnp.float32
        ),
        mesh=plsc.VectorSubcoreMesh(core_axis_name="c", subcore_axis_name="s"),
        scratch_types=[
            pltpu.VMEM((n_chunks, _CHUNK), jnp.int32),
            pltpu.VMEM((b_per_w, dim), jnp.float32),
            pltpu.VMEM((dim // 8, n_chunks, 8, _CHUNK), jnp.float32),
            [pltpu.SemaphoreType.DMA] * n_chunks,
        ],
        compiler_params=pltpu.CompilerParams(use_tc_tiling_on_sc=False, needs_layout_passes=False),
    )
    out4d = gather(idx, table)
    return out4d.transpose(1, 3, 0, 2).reshape(batch, dim)


# speedup vs baseline: 1.0095x; 1.0095x over previous
import functools

import jax
import jax.numpy as jnp
from jax import lax
from jax.experimental import pallas as pl
from jax.experimental.pallas import tpu as pltpu
from jax.experimental.pallas import tpu_sc as plsc

_NUM_CORES = 2
_NUM_SUBCORES = 16
_NUM_WORKERS = _NUM_CORES * _NUM_SUBCORES
_CHUNK = 128
_LANES = 16


def _gather_body(n_chunks, dim, b_per_w, idx_hbm, table_hbm, out_hbm,
                 idx_v, idxq_v, mq_v, chunks_v, buf_v, sems):
    wid = lax.axis_index("s") * _NUM_CORES + lax.axis_index("c")
    base = wid * b_per_w
    pltpu.sync_copy(idx_hbm.at[pl.ds(base, b_per_w)], idx_v)

    iota = lax.iota(jnp.int32, _LANES)
    n_lc = _CHUNK // _LANES
    for j in range(n_chunks):
        for lc in range(n_lc):
            v = idx_v[pl.ds(j * _CHUNK + lc * _LANES, _LANES)]
            idxq_v[j, pl.ds(lc * _LANES, _LANES)] = v >> 2
            mq_v[j, pl.ds(lc * _LANES, _LANES)] = (v & 3) << 5
    copies = [
        pltpu.async_copy(table_hbm.at[idxq_v.at[j]], chunks_v[j], sems[j])
        for j in range(n_chunks)
    ]
    for j in range(n_chunks):
        copies[j].wait()
        mvecs = [mq_v[j, pl.ds(lc * _LANES, _LANES)] for lc in range(n_lc)]
        l0 = [lc * _LANES + iota for lc in range(n_lc)]
        for c in range(dim):
            vals = [
                plsc.load_gather(chunks_v[j], [l0[lc], mvecs[lc] + c])
                for lc in range(n_lc)
            ]
            for lc in range(n_lc):
                buf_v[c // 8, j, c % 8, pl.ds(lc * _LANES, _LANES)] = vals[lc]
    j0 = wid * n_chunks
    for i in range(dim // 8):
        pltpu.sync_copy(buf_v.at[i], out_hbm.at[i, pl.ds(j0, n_chunks)])


def kernel(fuel_id, table):
    (batch,) = fuel_id.shape
    _, dim = table.shape
    b_per_w = batch // _NUM_WORKERS
    n_chunks = b_per_w // _CHUNK
    idx = fuel_id.astype(jnp.int32)
    tq = table.reshape(-1, 4 * dim)

    gather = pl.kernel(
        functools.partial(_gather_body, n_chunks, dim, b_per_w),
        out_type=jax.ShapeDtypeStruct(
            (dim // 8, batch // _CHUNK, 8, _CHUNK), jnp.float32
        ),
        mesh=plsc.VectorSubcoreMesh(core_axis_name="c", subcore_axis_name="s"),
        scratch_types=[
            pltpu.VMEM((b_per_w,), jnp.int32),
            pltpu.VMEM((n_chunks, _CHUNK), jnp.int32),
            pltpu.VMEM((n_chunks, _CHUNK), jnp.int32),
            [pltpu.VMEM((_CHUNK, 4 * dim), jnp.float32)] * n_chunks,
            pltpu.VMEM((dim // 8, n_chunks, 8, _CHUNK), jnp.float32),
            [pltpu.SemaphoreType.DMA] * n_chunks,
        ],
        compiler_params=pltpu.CompilerParams(
            use_tc_tiling_on_sc=True, needs_layout_passes=False
        ),
    )
    out4d = gather(idx, tq)
    return out4d.transpose(1, 3, 0, 2).reshape(batch, dim)
